# Initial kernel scaffold; baseline (speedup 1.0000x reference)
import functools

import jax
import jax.numpy as jnp
from jax import lax
from jax.experimental import pallas as pl
from jax.experimental.pallas import tpu as pltpu
from jax.experimental.pallas import tpu_sc as plsc

VOCAB = 1000
B_TOT = 4096 * 50
NUM_CORES = 2
NUM_SUBCORES = 16
NW = NUM_CORES * NUM_SUBCORES
B_PER_W = B_TOT // NW
CHUNK = 80
STEPS = B_PER_W // CHUNK


@functools.partial(
    pl.kernel,
    mesh=plsc.VectorSubcoreMesh(core_axis_name="c", subcore_axis_name="s"),
    out_type=jax.ShapeDtypeStruct((B_TOT, VOCAB), jnp.float32),
    scratch_types=[
        pltpu.VMEM((B_PER_W,), jnp.int32),
        pltpu.VMEM((CHUNK, VOCAB), jnp.float32),
        pltpu.SemaphoreType.DMA,
    ],
)
def _gather_rows(x_hbm, table_hbm, out_hbm, idx_v, rows_v, sem):
    wid = lax.axis_index("s") * NUM_CORES + lax.axis_index("c")
    base = wid * B_PER_W
    pltpu.sync_copy(x_hbm.at[pl.ds(base, B_PER_W)], idx_v)

    def step(i, carry):
        off = i * CHUNK
        pltpu.async_copy(
            table_hbm.at[idx_v.at[pl.ds(off, CHUNK)]], rows_v, sem
        ).wait()
        pltpu.sync_copy(rows_v, out_hbm.at[pl.ds(base + off, CHUNK)])
        return carry

    lax.fori_loop(0, STEPS, step, 0)


def kernel(x, bigram):
    xf = x.reshape(-1).astype(jnp.int32)
    out = _gather_rows(xf, bigram)
    return out.reshape(x.shape[0], x.shape[1], VOCAB)


# speedup vs baseline: 1.0269x; 1.0269x over previous
import functools

import jax
import jax.numpy as jnp
from jax import lax
from jax.experimental import pallas as pl
from jax.experimental.pallas import tpu as pltpu
from jax.experimental.pallas import tpu_sc as plsc

VOCAB = 1000
B_TOT = 4096 * 50
NUM_CORES = 2
NUM_SUBCORES = 16
NW = NUM_CORES * NUM_SUBCORES
B_PER_W = B_TOT // NW
CHUNK = 80
STEPS = B_PER_W // CHUNK


@functools.partial(
    pl.kernel,
    mesh=plsc.VectorSubcoreMesh(core_axis_name="c", subcore_axis_name="s"),
    compiler_params=pltpu.CompilerParams(use_tc_tiling_on_sc=False),
    out_type=jax.ShapeDtypeStruct((B_TOT, VOCAB), jnp.float32),
    scratch_types=[
        pltpu.VMEM((B_PER_W,), jnp.int32),
        pltpu.VMEM((CHUNK, VOCAB), jnp.float32),
        pltpu.SemaphoreType.DMA,
    ],
)
def _gather_rows(x_hbm, table_hbm, out_hbm, idx_v, rows_v, sem):
    wid = lax.axis_index("s") * NUM_CORES + lax.axis_index("c")
    base = wid * B_PER_W
    pltpu.sync_copy(x_hbm.at[pl.ds(base, B_PER_W)], idx_v)

    def step(i, carry):
        off = i * CHUNK
        pltpu.async_copy(
            table_hbm.at[idx_v.at[pl.ds(off, CHUNK)]], rows_v, sem
        ).wait()
        pltpu.sync_copy(rows_v, out_hbm.at[pl.ds(base + off, CHUNK)])
        return carry

    lax.fori_loop(0, STEPS, step, 0)


def kernel(x, bigram):
    xf = x.reshape(-1).astype(jnp.int32)
    out = _gather_rows(xf, bigram)
    return out.reshape(x.shape[0], x.shape[1], VOCAB)


# speedup vs baseline: 1.1621x; 1.1316x over previous
import functools

import jax
import jax.numpy as jnp
from jax import lax
from jax.experimental import pallas as pl
from jax.experimental.pallas import tpu as pltpu
from jax.experimental.pallas import tpu_sc as plsc

VOCAB = 1000
B_TOT = 4096 * 50
NUM_CORES = 2
NUM_SUBCORES = 16
NW = NUM_CORES * NUM_SUBCORES
B_PER_W = B_TOT // NW
CHUNK = 16
STEPS = B_PER_W // CHUNK
NPAIR = STEPS // 2


@functools.partial(
    pl.kernel,
    mesh=plsc.VectorSubcoreMesh(core_axis_name="c", subcore_axis_name="s"),
    compiler_params=pltpu.CompilerParams(use_tc_tiling_on_sc=False),
    out_type=jax.ShapeDtypeStruct((B_TOT, VOCAB), jnp.float32),
    scratch_types=[
        pltpu.VMEM_SHARED((VOCAB, VOCAB), jnp.float32),
        pltpu.VMEM((B_PER_W,), jnp.int32),
        pltpu.VMEM((CHUNK, VOCAB), jnp.float32),
        pltpu.VMEM((CHUNK, VOCAB), jnp.float32),
        pltpu.SemaphoreType.DMA,
        pltpu.SemaphoreType.DMA,
        pltpu.SemaphoreType.DMA,
        pltpu.SemaphoreType.DMA,
    ],
)
def _gather_rows(x_hbm, table_hbm, out_hbm, shared, idx_v, rows0, rows1,
                 sg0, sg1, sw0, sw1):
    cid = lax.axis_index("c")
    sid = lax.axis_index("s")
    wid = sid * NUM_CORES + cid
    base = wid * B_PER_W

    @pl.when(sid < 8)
    def _():
        pltpu.sync_copy(table_hbm.at[pl.ds(sid * 125, 125)],
                        shared.at[pl.ds(sid * 125, 125)])
    pltpu.sync_copy(x_hbm.at[pl.ds(base, B_PER_W)], idx_v)
    plsc.subcore_barrier()

    def gather(i, buf, sem):
        return pltpu.make_async_copy(
            shared.at[idx_v.at[pl.ds(i * CHUNK, CHUNK)]], buf, sem)

    def write(i, buf, sem):
        return pltpu.make_async_copy(
            buf, out_hbm.at[pl.ds(base + i * CHUNK, CHUNK)], sem)

    gather(0, rows0, sg0).start()

    def pair(g, carry):
        i0 = 2 * g
        i1 = i0 + 1

        @pl.when(g > 0)
        def _():
            write(i1 - 2, rows1, sw1).wait()
        gather(i1, rows1, sg1).start()
        gather(i0, rows0, sg0).wait()
        write(i0, rows0, sw0).start()
        gather(i1, rows1, sg1).wait()
        write(i0, rows0, sw0).wait()

        @pl.when(g < NPAIR - 1)
        def _():
            gather(i0 + 2, rows0, sg0).start()
        write(i1, rows1, sw1).start()
        return carry

    lax.fori_loop(0, NPAIR, pair, 0)
    write(STEPS - 1, rows1, sw1).wait()


def kernel(x, bigram):
    xf = x.reshape(-1).astype(jnp.int32)
    out = _gather_rows(xf, bigram)
    return out.reshape(x.shape[0], x.shape[1], VOCAB)
